# baseline (speedup 1.0000x reference)
import functools

import jax
import jax.numpy as jnp
from jax.experimental import pallas as pl
from jax.experimental.pallas import tpu as pltpu

_BF = jnp.bfloat16
_F32 = jnp.float32
_BI = 200


def _pad2(a, rows, cols, dtype):
    pr, pc = rows - a.shape[0], cols - a.shape[1]
    if pr or pc:
        a = jnp.pad(a, ((0, pr), (0, pc)))
    return a.astype(dtype)


def _rup(n, m):
    return (n + m - 1) // m * m


def _row_spec(bi, n):
    return pl.BlockSpec((bi, n), lambda i: (i, 0))


def _full_spec(shape):
    return pl.BlockSpec(shape, lambda i: (0, 0))


def _p1_body(a_ref, x_ref, w1_ref, w2_ref, adjb_ref, o_ref):
    ab = a_ref[...].astype(_BF)
    adjb_ref[...] = ab
    t = jnp.dot(ab, x_ref[...], preferred_element_type=_F32)
    t = jnp.dot(t.astype(_BF), w1_ref[...], preferred_element_type=_F32)
    t = jnp.maximum(t, 0.0).astype(_BF)
    o_ref[...] = jnp.dot(t, w2_ref[...], preferred_element_type=_F32).astype(_BF)


def _p2_body(a_ref, g_ref, o_ref):
    t = jnp.dot(a_ref[...], g_ref[...], preferred_element_type=_F32)
    o_ref[...] = jnp.maximum(t, 0.0).astype(_BF)


def _p3_body(a_ref, h_ref, w3_ref, w4_ref, o_ref, *, col_chunk):
    y = jnp.dot(a_ref[...], h_ref[...], preferred_element_type=_F32).astype(_BF)
    e3 = w3_ref.shape[1]
    acc = jnp.zeros((y.shape[0], w4_ref.shape[1]), _F32)
    for j in range(e3 // col_chunk):
        sl = slice(j * col_chunk, (j + 1) * col_chunk)
        t = jnp.dot(y, w3_ref[:, sl], preferred_element_type=_F32)
        t = jnp.maximum(t, 0.0).astype(_BF)
        acc = acc + jnp.dot(t, w4_ref[sl, :], preferred_element_type=_F32)
    o_ref[...] = acc.astype(_BF)


def _p4_body(a_ref, g_ref, h4_ref, rh4_ref):
    t = jnp.dot(a_ref[...], g_ref[...], preferred_element_type=_F32)
    h4_ref[...] = t
    rh4_ref[...] = jnp.maximum(t, 0.0).astype(_BF)


def _p5_body(a_ref, rh4_ref, h4_ref, w5_ref, clt_ref, fc1w_ref, fc1b_ref,
             fc2w_ref, fc2b_ref, xbar_ref, q_ref, pred_ref, *, n_real):
    s = jnp.dot(a_ref[...], rh4_ref[...], preferred_element_type=_F32)
    h5 = jnp.dot(s.astype(_BF), w5_ref[...], preferred_element_type=_F32)
    mask = jax.lax.broadcasted_iota(jnp.int32, h5.shape, 1) < n_real
    h5 = jnp.where(mask, h5, -1e30)
    m = jnp.max(h5, axis=1, keepdims=True)
    e = jnp.exp(h5 - m)
    pred_ref[...] = e / jnp.sum(e, axis=1, keepdims=True)

    h4 = h4_ref[...]
    clt = clt_ref[...]
    hh = jnp.sum(h4 * h4, axis=1, keepdims=True)
    cc = jnp.sum(clt * clt, axis=0, keepdims=True)
    hc = jax.lax.dot_general(h4, clt, (((1,), (0,)), ((), ())),
                             precision=jax.lax.Precision.HIGHEST,
                             preferred_element_type=_F32)
    d = hh - 2.0 * hc + cc
    qn = jnp.where(mask, 1.0 / (1.0 + d), 0.0)
    q_ref[...] = qn / jnp.sum(qn, axis=1, keepdims=True)

    rh4 = jnp.maximum(h4, 0.0).astype(_BF)
    deco = jnp.dot(rh4, fc1w_ref[...], preferred_element_type=_F32) + fc1b_ref[0:1, :]
    deco = jnp.maximum(deco, 0.0).astype(_BF)
    xb = jnp.dot(deco, fc2w_ref[...], preferred_element_type=_F32) + fc2b_ref[0:1, :]
    xbar_ref[...] = jnp.maximum(xb, 0.0)


def _params():
    return pltpu.CompilerParams(dimension_semantics=("parallel",))


def _mega_body(a_ref, h2_ref, w3_ref, w4_ref, w5_ref, clt_ref, fc1w_ref,
               fc1b_ref, fc2w_ref, fc2b_ref, xbar_ref, q_ref, pred_ref,
               s_ref, *, col_chunk, n_real, bi, nz):
    p = pl.program_id(0)
    i = pl.program_id(1)
    rows = pl.ds(i * bi, bi)

    @pl.when(p == 0)
    def _():
        y = jnp.dot(a_ref[...], h2_ref[...], preferred_element_type=_F32).astype(_BF)
        e3 = w3_ref.shape[1]
        acc = jnp.zeros((bi, nz), _F32)
        for j in range(e3 // col_chunk):
            sl = slice(j * col_chunk, (j + 1) * col_chunk)
            t = jnp.dot(y, w3_ref[:, sl], preferred_element_type=_F32)
            t = jnp.maximum(t, 0.0).astype(_BF)
            acc = acc + jnp.dot(t, w4_ref[sl, :], preferred_element_type=_F32)
        s_ref[rows, 0:nz] = acc

    @pl.when(p == 1)
    def _():
        g4 = s_ref[:, 0:nz].astype(_BF)
        t = jnp.dot(a_ref[...], g4, preferred_element_type=_F32)
        s_ref[rows, nz:2 * nz] = t

    @pl.when(p == 2)
    def _():
        rh4_all = jnp.maximum(s_ref[:, nz:2 * nz], 0.0).astype(_BF)
        s = jnp.dot(a_ref[...], rh4_all, preferred_element_type=_F32)
        h5 = jnp.dot(s.astype(_BF), w5_ref[...], preferred_element_type=_F32)
        mask = jax.lax.broadcasted_iota(jnp.int32, h5.shape, 1) < n_real
        h5 = jnp.where(mask, h5, -1e30)
        m = jnp.max(h5, axis=1, keepdims=True)
        e = jnp.exp(h5 - m)
        pred_ref[...] = e / jnp.sum(e, axis=1, keepdims=True)

        h4 = s_ref[rows, nz:2 * nz]
        clt = clt_ref[...]
        hh = jnp.sum(h4 * h4, axis=1, keepdims=True)
        cc = jnp.sum(clt * clt, axis=0, keepdims=True)
        hc = jax.lax.dot_general(h4, clt, (((1,), (0,)), ((), ())),
                                 precision=jax.lax.Precision.HIGHEST,
                                 preferred_element_type=_F32)
        d = hh - 2.0 * hc + cc
        qn = jnp.where(mask, 1.0 / (1.0 + d), 0.0)
        q_ref[...] = qn / jnp.sum(qn, axis=1, keepdims=True)

        rh4 = jnp.maximum(h4, 0.0).astype(_BF)
        deco = jnp.dot(rh4, fc1w_ref[...], preferred_element_type=_F32) + fc1b_ref[0:1, :]
        deco = jnp.maximum(deco, 0.0).astype(_BF)
        xb = jnp.dot(deco, fc2w_ref[...], preferred_element_type=_F32) + fc2b_ref[0:1, :]
        xbar_ref[...] = jnp.maximum(xb, 0.0)


def kernel(x, adj, W1, W2, W3, W4, W5, fc1_w, fc1_b, fc2_w, fc2_b, cluster):
    n, n_in = x.shape
    e1 = _rup(W1.shape[1], 128)
    e2 = _rup(W2.shape[1], 128)
    e3 = _rup(W3.shape[1], 128)
    nz = _rup(W4.shape[1], 16)
    nc = _rup(W5.shape[1], 16)
    ncl_real = cluster.shape[0]
    f1 = _rup(fc1_w.shape[1], 128)

    def _pick(pref):
        return pref if n % pref == 0 else (_BI if n % _BI == 0 else 8)

    bi = _pick(400)
    bi_c = _pick(1000)
    grid = (n // bi,)
    grid_c = (n // bi_c,)

    xb16 = x.astype(_BF)
    w1p = _pad2(W1, n_in, e1, _BF)
    w2p = _pad2(W2, e1, e2, _BF)
    w3p = _pad2(W3, e2, e3, _BF)
    w4p = _pad2(W4, e3, nz, _BF)
    w5p = _pad2(W5, nz, nc, _BF)
    fc1wp = _pad2(fc1_w, nz, f1, _BF)
    fc2wp = _pad2(fc2_w, f1, n_in, _BF)
    fc1bp = _pad2(fc1_b[None, :], 8, f1, _F32)
    fc2bp = _pad2(fc2_b[None, :], 8, n_in, _F32)
    cltp = _pad2(cluster.T, nz, nc, _F32)

    adj_b, g2 = pl.pallas_call(
        _p1_body, grid=grid,
        in_specs=[_row_spec(bi, n), _full_spec((n, n_in)),
                  _full_spec((n_in, e1)), _full_spec((e1, e2))],
        out_specs=[_row_spec(bi, n), _row_spec(bi, e2)],
        out_shape=[jax.ShapeDtypeStruct((n, n), _BF),
                   jax.ShapeDtypeStruct((n, e2), _BF)],
        compiler_params=pltpu.CompilerParams(
            dimension_semantics=("parallel",),
            vmem_limit_bytes=62 * 1024 * 1024),
    )(adj, xb16, w1p, w2p)

    def _adj_map(p, i):
        return (i, 0)

    def _const_map(p, i):
        return (0, 0)

    def _out_map(p, i):
        return (i, 0)

    h2 = pl.pallas_call(
        _p2_body, grid=grid_c,
        in_specs=[_row_spec(bi_c, n), _full_spec((n, e2))],
        out_specs=_row_spec(bi_c, e2),
        out_shape=jax.ShapeDtypeStruct((n, e2), _BF),
        compiler_params=_params(),
    )(adj_b, g2)

    bi_m = bi_c
    x_bar, q, pred = pl.pallas_call(
        functools.partial(_mega_body, col_chunk=256, n_real=ncl_real,
                          bi=bi_m, nz=nz),
        grid=(3, n // bi_m),
        in_specs=[pl.BlockSpec((bi_m, n), _adj_map),
                  pl.BlockSpec((n, e2), _const_map),
                  pl.BlockSpec((e2, e3), _const_map),
                  pl.BlockSpec((e3, nz), _const_map),
                  pl.BlockSpec((nz, nc), _const_map),
                  pl.BlockSpec((nz, nc), _const_map),
                  pl.BlockSpec((nz, f1), _const_map),
                  pl.BlockSpec((8, f1), _const_map),
                  pl.BlockSpec((f1, n_in), _const_map),
                  pl.BlockSpec((8, n_in), _const_map)],
        out_specs=[pl.BlockSpec((bi_m, n_in), _out_map),
                   pl.BlockSpec((bi_m, nc), _out_map),
                   pl.BlockSpec((bi_m, nc), _out_map)],
        out_shape=[jax.ShapeDtypeStruct((n, n_in), _F32),
                   jax.ShapeDtypeStruct((n, nc), _F32),
                   jax.ShapeDtypeStruct((n, nc), _F32)],
        scratch_shapes=[pltpu.VMEM((n, 2 * nz), _F32)],
        compiler_params=pltpu.CompilerParams(
            dimension_semantics=("arbitrary", "parallel"),
            vmem_limit_bytes=62 * 1024 * 1024),
    )(adj_b, h2, w3p, w4p, w5p, cltp, fc1wp, fc1bp, fc2wp, fc2bp)

    return (x_bar, q[:, :ncl_real], pred[:, :ncl_real])


# speedup vs baseline: 1.0008x; 1.0008x over previous
import functools

import jax
import jax.numpy as jnp
from jax.experimental import pallas as pl
from jax.experimental.pallas import tpu as pltpu

_BF = jnp.bfloat16
_F32 = jnp.float32
_BI = 200


def _pad2(a, rows, cols, dtype):
    pr, pc = rows - a.shape[0], cols - a.shape[1]
    if pr or pc:
        a = jnp.pad(a, ((0, pr), (0, pc)))
    return a.astype(dtype)


def _rup(n, m):
    return (n + m - 1) // m * m


def _row_spec(bi, n):
    return pl.BlockSpec((bi, n), lambda i: (i, 0))


def _full_spec(shape):
    return pl.BlockSpec(shape, lambda i: (0, 0))


def _p1_body(a_ref, x_ref, w1_ref, w2_ref, adjb_ref, o_ref):
    ab = a_ref[...].astype(_BF)
    adjb_ref[...] = ab
    t = jnp.dot(ab, x_ref[...], preferred_element_type=_F32)
    t = jnp.dot(t.astype(_BF), w1_ref[...], preferred_element_type=_F32)
    t = jnp.maximum(t, 0.0).astype(_BF)
    o_ref[...] = jnp.dot(t, w2_ref[...], preferred_element_type=_F32).astype(_BF)


def _p2_body(a_ref, g_ref, o_ref):
    t = jnp.dot(a_ref[...], g_ref[...], preferred_element_type=_F32)
    o_ref[...] = jnp.maximum(t, 0.0).astype(_BF)


def _p3_body(a_ref, h_ref, w3_ref, w4_ref, o_ref, *, col_chunk):
    y = jnp.dot(a_ref[...], h_ref[...], preferred_element_type=_F32).astype(_BF)
    e3 = w3_ref.shape[1]
    acc = jnp.zeros((y.shape[0], w4_ref.shape[1]), _F32)
    for j in range(e3 // col_chunk):
        sl = slice(j * col_chunk, (j + 1) * col_chunk)
        t = jnp.dot(y, w3_ref[:, sl], preferred_element_type=_F32)
        t = jnp.maximum(t, 0.0).astype(_BF)
        acc = acc + jnp.dot(t, w4_ref[sl, :], preferred_element_type=_F32)
    o_ref[...] = acc.astype(_BF)


def _p4_body(a_ref, g_ref, h4_ref, rh4_ref):
    t = jnp.dot(a_ref[...], g_ref[...], preferred_element_type=_F32)
    h4_ref[...] = t
    rh4_ref[...] = jnp.maximum(t, 0.0).astype(_BF)


def _p5_body(a_ref, rh4_ref, h4_ref, w5_ref, clt_ref, fc1w_ref, fc1b_ref,
             fc2w_ref, fc2b_ref, xbar_ref, q_ref, pred_ref, *, n_real):
    s = jnp.dot(a_ref[...], rh4_ref[...], preferred_element_type=_F32)
    h5 = jnp.dot(s.astype(_BF), w5_ref[...], preferred_element_type=_F32)
    mask = jax.lax.broadcasted_iota(jnp.int32, h5.shape, 1) < n_real
    h5 = jnp.where(mask, h5, -1e30)
    m = jnp.max(h5, axis=1, keepdims=True)
    e = jnp.exp(h5 - m)
    pred_ref[...] = e / jnp.sum(e, axis=1, keepdims=True)

    h4 = h4_ref[...]
    clt = clt_ref[...]
    hh = jnp.sum(h4 * h4, axis=1, keepdims=True)
    cc = jnp.sum(clt * clt, axis=0, keepdims=True)
    hc = jax.lax.dot_general(h4, clt, (((1,), (0,)), ((), ())),
                             precision=jax.lax.Precision.HIGHEST,
                             preferred_element_type=_F32)
    d = hh - 2.0 * hc + cc
    qn = jnp.where(mask, 1.0 / (1.0 + d), 0.0)
    q_ref[...] = qn / jnp.sum(qn, axis=1, keepdims=True)

    rh4 = jnp.maximum(h4, 0.0).astype(_BF)
    deco = jnp.dot(rh4, fc1w_ref[...], preferred_element_type=_F32) + fc1b_ref[0:1, :]
    deco = jnp.maximum(deco, 0.0).astype(_BF)
    xb = jnp.dot(deco, fc2w_ref[...], preferred_element_type=_F32) + fc2b_ref[0:1, :]
    xbar_ref[...] = jnp.maximum(xb, 0.0)


def _params():
    return pltpu.CompilerParams(dimension_semantics=("parallel",))


def _mega_body(a_ref, h2_ref, w3_ref, w4_ref, w5_ref, clt_ref, fc1w_ref,
               fc1b_ref, fc2w_ref, fc2b_ref, xbar_ref, q_ref, pred_ref,
               s_ref, *, col_chunk, n_real, bi, nz):
    p = pl.program_id(0)
    i = pl.program_id(1)
    rows = pl.ds(i * bi, bi)

    @pl.when(p == 0)
    def _():
        y = jnp.dot(a_ref[...], h2_ref[...], preferred_element_type=_F32).astype(_BF)
        e3 = w3_ref.shape[1]
        acc = jnp.zeros((bi, nz), _F32)
        for j in range(e3 // col_chunk):
            sl = slice(j * col_chunk, (j + 1) * col_chunk)
            t = jnp.dot(y, w3_ref[:, sl], preferred_element_type=_F32)
            t = jnp.maximum(t, 0.0).astype(_BF)
            acc = acc + jnp.dot(t, w4_ref[sl, :], preferred_element_type=_F32)
        s_ref[rows, 0:nz] = acc

    @pl.when(p == 1)
    def _():
        g4 = s_ref[:, 0:nz].astype(_BF)
        t = jnp.dot(a_ref[...], g4, preferred_element_type=_F32)
        s_ref[rows, nz:2 * nz] = t

    @pl.when(p == 2)
    def _():
        rh4_all = jnp.maximum(s_ref[:, nz:2 * nz], 0.0).astype(_BF)
        s = jnp.dot(a_ref[...], rh4_all, preferred_element_type=_F32)
        h5 = jnp.dot(s.astype(_BF), w5_ref[...], preferred_element_type=_F32)
        mask = jax.lax.broadcasted_iota(jnp.int32, h5.shape, 1) < n_real
        h5 = jnp.where(mask, h5, -1e30)
        m = jnp.max(h5, axis=1, keepdims=True)
        e = jnp.exp(h5 - m)
        pred_ref[...] = e / jnp.sum(e, axis=1, keepdims=True)

        h4 = s_ref[rows, nz:2 * nz]
        clt = clt_ref[...]
        hh = jnp.sum(h4 * h4, axis=1, keepdims=True)
        cc = jnp.sum(clt * clt, axis=0, keepdims=True)
        hc = jax.lax.dot_general(h4, clt, (((1,), (0,)), ((), ())),
                                 precision=jax.lax.Precision.HIGHEST,
                                 preferred_element_type=_F32)
        d = hh - 2.0 * hc + cc
        qn = jnp.where(mask, 1.0 / (1.0 + d), 0.0)
        q_ref[...] = qn / jnp.sum(qn, axis=1, keepdims=True)

        rh4 = jnp.maximum(h4, 0.0).astype(_BF)
        deco = jnp.dot(rh4, fc1w_ref[...], preferred_element_type=_F32) + fc1b_ref[0:1, :]
        deco = jnp.maximum(deco, 0.0).astype(_BF)
        xb = jnp.dot(deco, fc2w_ref[...], preferred_element_type=_F32) + fc2b_ref[0:1, :]
        xbar_ref[...] = jnp.maximum(xb, 0.0)


def kernel(x, adj, W1, W2, W3, W4, W5, fc1_w, fc1_b, fc2_w, fc2_b, cluster):
    n, n_in = x.shape
    e1 = _rup(W1.shape[1], 128)
    e2 = _rup(W2.shape[1], 128)
    e3 = _rup(W3.shape[1], 128)
    nz = _rup(W4.shape[1], 16)
    nc = _rup(W5.shape[1], 16)
    ncl_real = cluster.shape[0]
    f1 = _rup(fc1_w.shape[1], 128)

    def _pick(pref):
        return pref if n % pref == 0 else (_BI if n % _BI == 0 else 8)

    bi = _pick(400)
    bi_c = _pick(1000)
    grid = (n // bi,)
    grid_c = (n // bi_c,)

    xb16 = x.astype(_BF)
    w1p = _pad2(W1, n_in, e1, _BF)
    w2p = _pad2(W2, e1, e2, _BF)
    w3p = _pad2(W3, e2, e3, _BF)
    w4p = _pad2(W4, e3, nz, _BF)
    w5p = _pad2(W5, nz, nc, _BF)
    fc1wp = _pad2(fc1_w, nz, f1, _BF)
    fc2wp = _pad2(fc2_w, f1, n_in, _BF)
    fc1bp = _pad2(fc1_b[None, :], 8, f1, _F32)
    fc2bp = _pad2(fc2_b[None, :], 8, n_in, _F32)
    cltp = _pad2(cluster.T, nz, nc, _F32)

    adj_b, g2 = pl.pallas_call(
        _p1_body, grid=grid,
        in_specs=[_row_spec(bi, n), _full_spec((n, n_in)),
                  _full_spec((n_in, e1)), _full_spec((e1, e2))],
        out_specs=[_row_spec(bi, n), _row_spec(bi, e2)],
        out_shape=[jax.ShapeDtypeStruct((n, n), _BF),
                   jax.ShapeDtypeStruct((n, e2), _BF)],
        compiler_params=pltpu.CompilerParams(
            dimension_semantics=("parallel",),
            vmem_limit_bytes=62 * 1024 * 1024),
    )(adj, xb16, w1p, w2p)

    def _adj_map(p, i):
        return (i, 0)

    def _const_map(p, i):
        return (0, 0)

    def _out_map(p, i):
        return (i, 0)

    h2 = pl.pallas_call(
        _p2_body, grid=grid_c,
        in_specs=[_row_spec(bi_c, n), _full_spec((n, e2))],
        out_specs=_row_spec(bi_c, e2),
        out_shape=jax.ShapeDtypeStruct((n, e2), _BF),
        compiler_params=pltpu.CompilerParams(
            dimension_semantics=("parallel",),
            vmem_limit_bytes=62 * 1024 * 1024),
    )(adj_b, g2)

    bi_m = bi_c
    x_bar, q, pred = pl.pallas_call(
        functools.partial(_mega_body, col_chunk=256, n_real=ncl_real,
                          bi=bi_m, nz=nz),
        grid=(3, n // bi_m),
        in_specs=[pl.BlockSpec((bi_m, n), _adj_map),
                  pl.BlockSpec((n, e2), _const_map),
                  pl.BlockSpec((e2, e3), _const_map),
                  pl.BlockSpec((e3, nz), _const_map),
                  pl.BlockSpec((nz, nc), _const_map),
                  pl.BlockSpec((nz, nc), _const_map),
                  pl.BlockSpec((nz, f1), _const_map),
                  pl.BlockSpec((8, f1), _const_map),
                  pl.BlockSpec((f1, n_in), _const_map),
                  pl.BlockSpec((8, n_in), _const_map)],
        out_specs=[pl.BlockSpec((bi_m, n_in), _out_map),
                   pl.BlockSpec((bi_m, nc), _out_map),
                   pl.BlockSpec((bi_m, nc), _out_map)],
        out_shape=[jax.ShapeDtypeStruct((n, n_in), _F32),
                   jax.ShapeDtypeStruct((n, nc), _F32),
                   jax.ShapeDtypeStruct((n, nc), _F32)],
        scratch_shapes=[pltpu.VMEM((n, 2 * nz), _F32)],
        compiler_params=pltpu.CompilerParams(
            dimension_semantics=("arbitrary", "parallel"),
            vmem_limit_bytes=62 * 1024 * 1024),
    )(adj_b, h2, w3p, w4p, w5p, cltp, fc1wp, fc1bp, fc2wp, fc2bp)

    return (x_bar, q[:, :ncl_real], pred[:, :ncl_real])
